# baseline (speedup 1.0000x reference)
import functools

import numpy as np

import jax
import jax.numpy as jnp
from jax.experimental import pallas as pl
from jax.experimental.pallas import tpu as pltpu


def _fused_net_kernel(x_ref, ws_ref, bs_ref, we_ref, be_ref,
                      wdw_ref, bdw_ref, w1_ref, b1_ref, w2_ref, b2_ref,
                      wp_ref, bp_ref, wh_ref, bh_ref, wf_ref, bf_ref,
                      o_ref, *, Ho, Wo):
    S = Ho * Wo
    inv_s = 1.0 / S

    x = x_ref[0]
    L = x.shape[-1]
    zr = jnp.zeros((1, Wo, L), jnp.bfloat16)
    zc = jnp.zeros((Ho, 1, L), jnp.bfloat16)
    xs = jnp.concatenate([x[:, 1:, :], zc], axis=1)
    xd = jnp.concatenate([x[1:, :, :], zr], axis=0)
    xds = jnp.concatenate([xs[1:, :, :], zr], axis=0)
    cols = jnp.concatenate([x, xs, xd, xds], axis=-1).reshape(S, 4 * L)
    h = jnp.dot(cols, ws_ref[...],
                preferred_element_type=jnp.float32) + bs_ref[...]
    h = h * jax.nn.sigmoid(h)
    hb = h.astype(jnp.bfloat16)

    e = jnp.dot(hb, we_ref[...],
                preferred_element_type=jnp.float32) + be_ref[...]
    e = e * jax.nn.sigmoid(e)
    C2 = we_ref.shape[1]
    eb = e.astype(jnp.bfloat16).reshape(Ho, Wo, C2)

    zr2 = jnp.zeros((1, Wo, C2), jnp.bfloat16)
    ep = jnp.concatenate([zr2, eb, zr2], axis=0)
    zc2 = jnp.zeros((Ho + 2, 1, C2), jnp.bfloat16)
    ep = jnp.concatenate([zc2, ep, zc2], axis=1)
    shifted = [ep[:, j:j + Wo, :] for j in range(3)]
    acc = jnp.zeros((Ho, Wo, C2), jnp.float32)
    for i in range(3):
        for j in range(3):
            acc = acc + (shifted[j][i:i + Ho] * wdw_ref[3 * i + j]
                         ).astype(jnp.float32)
    y = acc + bdw_ref[...]
    y = y * jax.nn.sigmoid(y)

    pooled = jnp.sum(jnp.sum(y, axis=0), axis=0, keepdims=True) * inv_s
    r = jnp.dot(pooled.astype(jnp.bfloat16), w1_ref[...],
                preferred_element_type=jnp.float32) + b1_ref[...]
    r = r * jax.nn.sigmoid(r)
    g = jax.nn.sigmoid(jnp.dot(r.astype(jnp.bfloat16), w2_ref[...],
                               preferred_element_type=jnp.float32)
                       + b2_ref[...])

    dg = (y.astype(jnp.bfloat16).reshape(S, C2) * g).astype(jnp.bfloat16)
    hn = (jnp.dot(dg, wp_ref[...], preferred_element_type=jnp.float32)
          + bp_ref[...] + hb.astype(jnp.float32))
    hd = (jnp.dot(hn.astype(jnp.bfloat16), wh_ref[...],
                  preferred_element_type=jnp.float32) + bh_ref[...])
    hd = hd * jax.nn.sigmoid(hd)
    p2 = jnp.sum(hd, axis=0, keepdims=True) * inv_s
    logits = (jnp.dot(p2.astype(jnp.bfloat16), wf_ref[...],
                      preferred_element_type=jnp.float32) + bf_ref[...])
    o_ref[0] = logits


def _blockdiag2(w):
    K, N = w.shape
    z = jnp.zeros((K, N), w.dtype)
    return jnp.concatenate(
        [jnp.concatenate([w, z], axis=1), jnp.concatenate([z, w], axis=1)],
        axis=0)


def _pair2(v):
    return jnp.tile(v.reshape(1, -1).astype(jnp.float32), (1, 2))


def _stem_s2d_weights(stem_w, c_in):
    cs = stem_w.shape[1]
    src = np.zeros((4, 4 * c_in), np.int32)
    msk = np.zeros((4, 4 * c_in), np.float32)
    for bi in range(2):
        for bj in range(2):
            for pi in range(2):
                for pj in range(2):
                    i, j = 2 * bi + pi, 2 * bj + pj
                    if i < 3 and j < 3:
                        for c in range(c_in):
                            src[bi * 2 + bj, (pi * 2 + pj) * c_in + c] = \
                                (i * 3 + j) * c_in + c
                            msk[bi * 2 + bj, (pi * 2 + pj) * c_in + c] = 1.0
    w = stem_w[src.reshape(-1)].reshape(4, 4 * c_in, cs)
    return w * jnp.asarray(msk, stem_w.dtype)[:, :, None]


def kernel(x_nchw, stem_w, stem_b, exp_w, exp_b, dw_w, dw_b,
           se_r_w, se_r_b, se_e_w, se_e_b, proj_w, proj_b,
           head_w, head_b, fc_w, fc_b):
    B, C_IN, H, W = x_nchw.shape
    Ho, Wo = H // 2, W // 2
    S = Ho * Wo
    P = B // 2
    Cs = stem_w.shape[1]
    C = exp_w.shape[1]
    Cse = se_r_w.shape[1]
    Ch = head_w.shape[1]
    NC = fc_w.shape[1]
    Lp = 2 * 4 * C_IN

    xr = x_nchw.reshape(P, 2, C_IN, Ho, 2, Wo, 2)
    sd = jnp.transpose(xr, (0, 3, 5, 1, 4, 6, 2)).reshape(
        P, Ho, Wo, Lp).astype(jnp.bfloat16)

    wt = _stem_s2d_weights(stem_w, C_IN)
    zt = jnp.zeros_like(wt)
    ws2 = jnp.concatenate(
        [jnp.concatenate([wt, zt], axis=2),
         jnp.concatenate([zt, wt], axis=2)], axis=1).reshape(4 * Lp, 2 * Cs)

    we2 = _blockdiag2(exp_w)
    wdw2 = jnp.tile(dw_w, (1, 2))
    w1_2 = _blockdiag2(se_r_w)
    w2_2 = _blockdiag2(se_e_w)
    wp2 = _blockdiag2(proj_w)
    wh2 = _blockdiag2(head_w)
    wf2 = _blockdiag2(fc_w)

    out = pl.pallas_call(
        functools.partial(_fused_net_kernel, Ho=Ho, Wo=Wo),
        out_shape=jax.ShapeDtypeStruct((P, 1, 2 * NC), jnp.float32),
        grid=(P,),
        in_specs=[
            pl.BlockSpec((1, Ho, Wo, Lp), lambda b: (b, 0, 0, 0)),
            pl.BlockSpec((4 * Lp, 2 * Cs), lambda b: (0, 0)),
            pl.BlockSpec((1, 2 * Cs), lambda b: (0, 0)),
            pl.BlockSpec((2 * Cs, 2 * C), lambda b: (0, 0)),
            pl.BlockSpec((1, 2 * C), lambda b: (0, 0)),
            pl.BlockSpec((9, 2 * C), lambda b: (0, 0)),
            pl.BlockSpec((1, 2 * C), lambda b: (0, 0)),
            pl.BlockSpec((2 * C, 2 * Cse), lambda b: (0, 0)),
            pl.BlockSpec((1, 2 * Cse), lambda b: (0, 0)),
            pl.BlockSpec((2 * Cse, 2 * C), lambda b: (0, 0)),
            pl.BlockSpec((1, 2 * C), lambda b: (0, 0)),
            pl.BlockSpec((2 * C, 2 * Cs), lambda b: (0, 0)),
            pl.BlockSpec((1, 2 * Cs), lambda b: (0, 0)),
            pl.BlockSpec((2 * Cs, 2 * Ch), lambda b: (0, 0)),
            pl.BlockSpec((1, 2 * Ch), lambda b: (0, 0)),
            pl.BlockSpec((2 * Ch, 2 * NC), lambda b: (0, 0)),
            pl.BlockSpec((1, 2 * NC), lambda b: (0, 0)),
        ],
        out_specs=pl.BlockSpec((1, 1, 2 * NC), lambda b: (b, 0, 0)),
        compiler_params=pltpu.CompilerParams(dimension_semantics=("parallel",)),
    )(sd, ws2, _pair2(stem_b), we2, _pair2(exp_b),
      wdw2, _pair2(dw_b), w1_2, _pair2(se_r_b), w2_2, _pair2(se_e_b),
      wp2, _pair2(proj_b), wh2, _pair2(head_b), wf2, _pair2(fc_b))
    return out.reshape(B, NC)


# speedup vs baseline: 1.6928x; 1.6928x over previous
import functools

import jax
import jax.numpy as jnp
from jax.experimental import pallas as pl
from jax.experimental.pallas import tpu as pltpu


def _fused_net_kernel(cols_ref, ws_ref, bs_ref, we_ref, be_ref,
                      wdw_ref, bdw_ref, w1_ref, b1_ref, w2_ref, b2_ref,
                      wp_ref, bp_ref, wh_ref, bh_ref, wf_ref, bf_ref,
                      o_ref, *, Ho, Wo):
    S = Ho * Wo
    inv_s = 1.0 / S

    cols = cols_ref[0]
    h = jnp.dot(cols, ws_ref[...],
                preferred_element_type=jnp.float32) + bs_ref[...]
    h = h * jax.nn.sigmoid(h)
    hb = h.astype(jnp.bfloat16)

    e = jnp.dot(hb, we_ref[...],
                preferred_element_type=jnp.float32) + be_ref[...]
    e = e * jax.nn.sigmoid(e)
    C2 = we_ref.shape[1]
    eb = e.astype(jnp.bfloat16).reshape(Ho, Wo, C2)

    zr2 = jnp.zeros((1, Wo, C2), jnp.bfloat16)
    epv = jnp.concatenate([zr2, eb, zr2], axis=0)
    zc2 = jnp.zeros((Ho + 2, 1, C2), jnp.bfloat16)
    w16 = wdw_ref[...]
    taps = []
    for j in range(3):
        if j == 0:
            sj = jnp.concatenate([zc2, epv[:, :Wo - 1, :]], axis=1)
        elif j == 1:
            sj = epv
        else:
            sj = jnp.concatenate([epv[:, 1:, :], zc2], axis=1)
        for i in range(3):
            taps.append(sj[i:i + Ho] * w16[3 * i + j])
    while len(taps) > 1:
        taps = [taps[k] + taps[k + 1] if k + 1 < len(taps) else taps[k]
                for k in range(0, len(taps), 2)]
    y = taps[0].astype(jnp.float32) + bdw_ref[...]
    y = y * jax.nn.sigmoid(y)

    pooled = jnp.sum(jnp.sum(y, axis=0), axis=0, keepdims=True) * inv_s
    r = jnp.dot(pooled.astype(jnp.bfloat16), w1_ref[...],
                preferred_element_type=jnp.float32) + b1_ref[...]
    r = r * jax.nn.sigmoid(r)
    g = jax.nn.sigmoid(jnp.dot(r.astype(jnp.bfloat16), w2_ref[...],
                               preferred_element_type=jnp.float32)
                       + b2_ref[...])

    dg = (y.astype(jnp.bfloat16).reshape(S, C2) * g).astype(jnp.bfloat16)
    hn = (jnp.dot(dg, wp_ref[...], preferred_element_type=jnp.float32)
          + bp_ref[...] + hb.astype(jnp.float32))
    hd = (jnp.dot(hn.astype(jnp.bfloat16), wh_ref[...],
                  preferred_element_type=jnp.float32) + bh_ref[...])
    hd = hd * jax.nn.sigmoid(hd)
    p2 = jnp.sum(hd, axis=0, keepdims=True) * inv_s
    logits = (jnp.dot(p2.astype(jnp.bfloat16), wf_ref[...],
                      preferred_element_type=jnp.float32) + bf_ref[...])
    o_ref[0] = logits


def _blockdiag2(w):
    K, N = w.shape
    z = jnp.zeros((K, N), w.dtype)
    return jnp.concatenate(
        [jnp.concatenate([w, z], axis=1), jnp.concatenate([z, w], axis=1)],
        axis=0)


def _pair2(v):
    return jnp.tile(v.reshape(1, -1).astype(jnp.float32), (1, 2))


def kernel(x_nchw, stem_w, stem_b, exp_w, exp_b, dw_w, dw_b,
           se_r_w, se_r_b, se_e_w, se_e_b, proj_w, proj_b,
           head_w, head_b, fc_w, fc_b):
    B, C_IN, H, W = x_nchw.shape
    Ho, Wo = H // 2, W // 2
    S = Ho * Wo
    P = B // 2
    Cs = stem_w.shape[1]
    C = exp_w.shape[1]
    Cse = se_r_w.shape[1]
    Ch = head_w.shape[1]
    NC = fc_w.shape[1]

    x = jnp.transpose(x_nchw, (0, 2, 3, 1)).astype(jnp.bfloat16)
    xp = jnp.pad(x, ((0, 0), (0, 1), (0, 1), (0, 0)))
    x2 = jnp.concatenate([xp[0::2], xp[1::2]], axis=-1)
    taps = [x2[:, i:i + 2 * Ho:2, j:j + 2 * Wo:2, :]
            for i in range(3) for j in range(3)]
    cols = jnp.concatenate(taps, axis=-1).reshape(P, S, 9 * 2 * C_IN)

    ws3 = stem_w.reshape(9, C_IN, Cs)
    z3 = jnp.zeros_like(ws3)
    ws2 = jnp.concatenate(
        [jnp.concatenate([ws3, z3], axis=2),
         jnp.concatenate([z3, ws3], axis=2)], axis=1).reshape(9 * 2 * C_IN,
                                                              2 * Cs)
    we2 = _blockdiag2(exp_w)
    wdw2 = jnp.tile(dw_w, (1, 2))
    w1_2 = _blockdiag2(se_r_w)
    w2_2 = _blockdiag2(se_e_w)
    wp2 = _blockdiag2(proj_w)
    wh2 = _blockdiag2(head_w)
    wf2 = _blockdiag2(fc_w)

    out = pl.pallas_call(
        functools.partial(_fused_net_kernel, Ho=Ho, Wo=Wo),
        out_shape=jax.ShapeDtypeStruct((P, 1, 2 * NC), jnp.float32),
        grid=(P,),
        in_specs=[
            pl.BlockSpec((1, S, 9 * 2 * C_IN), lambda b: (b, 0, 0)),
            pl.BlockSpec((9 * 2 * C_IN, 2 * Cs), lambda b: (0, 0)),
            pl.BlockSpec((1, 2 * Cs), lambda b: (0, 0)),
            pl.BlockSpec((2 * Cs, 2 * C), lambda b: (0, 0)),
            pl.BlockSpec((1, 2 * C), lambda b: (0, 0)),
            pl.BlockSpec((9, 2 * C), lambda b: (0, 0)),
            pl.BlockSpec((1, 2 * C), lambda b: (0, 0)),
            pl.BlockSpec((2 * C, 2 * Cse), lambda b: (0, 0)),
            pl.BlockSpec((1, 2 * Cse), lambda b: (0, 0)),
            pl.BlockSpec((2 * Cse, 2 * C), lambda b: (0, 0)),
            pl.BlockSpec((1, 2 * C), lambda b: (0, 0)),
            pl.BlockSpec((2 * C, 2 * Cs), lambda b: (0, 0)),
            pl.BlockSpec((1, 2 * Cs), lambda b: (0, 0)),
            pl.BlockSpec((2 * Cs, 2 * Ch), lambda b: (0, 0)),
            pl.BlockSpec((1, 2 * Ch), lambda b: (0, 0)),
            pl.BlockSpec((2 * Ch, 2 * NC), lambda b: (0, 0)),
            pl.BlockSpec((1, 2 * NC), lambda b: (0, 0)),
        ],
        out_specs=pl.BlockSpec((1, 1, 2 * NC), lambda b: (b, 0, 0)),
        compiler_params=pltpu.CompilerParams(dimension_semantics=("parallel",)),
    )(cols, ws2, _pair2(stem_b), we2, _pair2(exp_b),
      wdw2, _pair2(dw_b), w1_2, _pair2(se_r_b), w2_2, _pair2(se_e_b),
      wp2, _pair2(proj_b), wh2, _pair2(head_b), wf2, _pair2(fc_b))
    return out.reshape(P, 2, NC).reshape(B, NC)


# speedup vs baseline: 6.0644x; 3.5824x over previous
import functools

import numpy as np

import jax
import jax.numpy as jnp
from jax.experimental import pallas as pl
from jax.experimental.pallas import tpu as pltpu


def _fused_net_kernel(cols_ref, ws_ref, bs_ref, we_ref, be_ref,
                      wdw_ref, bdw_ref, w1_ref, b1_ref, w2_ref, b2_ref,
                      wp_ref, bp_ref, wh_ref, bh_ref, wf_ref, bf_ref,
                      o_ref, *, Ho, Wo):
    S = Ho * Wo
    inv_s = 1.0 / S

    cols = cols_ref[0]
    h = jnp.dot(cols, ws_ref[...],
                preferred_element_type=jnp.float32) + bs_ref[...]
    h = h * jax.nn.sigmoid(h)
    hb = h.astype(jnp.bfloat16)

    e = jnp.dot(hb, we_ref[...],
                preferred_element_type=jnp.float32) + be_ref[...]
    e = e * jax.nn.sigmoid(e)
    C2 = we_ref.shape[1]
    eb = e.astype(jnp.bfloat16).reshape(Ho, Wo, C2)

    zr2 = jnp.zeros((1, Wo, C2), jnp.bfloat16)
    epv = jnp.concatenate([zr2, eb, zr2], axis=0)
    zc2 = jnp.zeros((Ho + 2, 1, C2), jnp.bfloat16)
    w16 = wdw_ref[...]
    taps = []
    for j in range(3):
        if j == 0:
            sj = jnp.concatenate([zc2, epv[:, :Wo - 1, :]], axis=1)
        elif j == 1:
            sj = epv
        else:
            sj = jnp.concatenate([epv[:, 1:, :], zc2], axis=1)
        for i in range(3):
            taps.append(sj[i:i + Ho] * w16[3 * i + j])
    while len(taps) > 1:
        taps = [taps[k] + taps[k + 1] if k + 1 < len(taps) else taps[k]
                for k in range(0, len(taps), 2)]
    y = taps[0].astype(jnp.float32) + bdw_ref[...]
    y = y * jax.nn.sigmoid(y)

    pooled = jnp.sum(jnp.sum(y, axis=0), axis=0, keepdims=True) * inv_s
    r = jnp.dot(pooled.astype(jnp.bfloat16), w1_ref[...],
                preferred_element_type=jnp.float32) + b1_ref[...]
    r = r * jax.nn.sigmoid(r)
    g = jax.nn.sigmoid(jnp.dot(r.astype(jnp.bfloat16), w2_ref[...],
                               preferred_element_type=jnp.float32)
                       + b2_ref[...])

    dg = (y.astype(jnp.bfloat16).reshape(S, C2) * g).astype(jnp.bfloat16)
    hn = (jnp.dot(dg, wp_ref[...], preferred_element_type=jnp.float32)
          + bp_ref[...] + hb.astype(jnp.float32))
    hd = (jnp.dot(hn.astype(jnp.bfloat16), wh_ref[...],
                  preferred_element_type=jnp.float32) + bh_ref[...])
    hd = hd * jax.nn.sigmoid(hd)
    p2 = jnp.sum(hd, axis=0, keepdims=True) * inv_s
    logits = (jnp.dot(p2.astype(jnp.bfloat16), wf_ref[...],
                      preferred_element_type=jnp.float32) + bf_ref[...])
    o_ref[0] = logits


def _blockdiag2(w):
    K, N = w.shape
    z = jnp.zeros((K, N), w.dtype)
    return jnp.concatenate(
        [jnp.concatenate([w, z], axis=1), jnp.concatenate([z, w], axis=1)],
        axis=0)


def _pair2(v):
    return jnp.tile(v.reshape(1, -1).astype(jnp.float32), (1, 2))


def kernel(x_nchw, stem_w, stem_b, exp_w, exp_b, dw_w, dw_b,
           se_r_w, se_r_b, se_e_w, se_e_b, proj_w, proj_b,
           head_w, head_b, fc_w, fc_b):
    B, C_IN, H, W = x_nchw.shape
    Ho, Wo = H // 2, W // 2
    S = Ho * Wo
    P = B // 2
    Cs = stem_w.shape[1]
    C = exp_w.shape[1]
    Cse = se_r_w.shape[1]
    Ch = head_w.shape[1]
    NC = fc_w.shape[1]

    xr = x_nchw.reshape(P, 2 * C_IN, H, W).astype(jnp.bfloat16)
    cols = jax.lax.conv_general_dilated_patches(
        xr, (3, 3), (2, 2), [(0, 1), (0, 1)],
        dimension_numbers=('NCHW', 'HWIO', 'NHWC')).reshape(P, S,
                                                            9 * 2 * C_IN)

    src = np.zeros(9 * 2 * C_IN, np.int32)
    blk = np.zeros(9 * 2 * C_IN, np.float32)
    for cc in range(2 * C_IN):
        img, corig = divmod(cc, C_IN)
        for t in range(9):
            src[cc * 9 + t] = t * C_IN + corig
            blk[cc * 9 + t] = float(img)
    w54 = stem_w[src]
    m1 = jnp.asarray(blk, stem_w.dtype)[:, None]
    ws2 = jnp.concatenate([w54 * (1.0 - m1), w54 * m1], axis=1)
    we2 = _blockdiag2(exp_w)
    wdw2 = jnp.tile(dw_w, (1, 2))
    w1_2 = _blockdiag2(se_r_w)
    w2_2 = _blockdiag2(se_e_w)
    wp2 = _blockdiag2(proj_w)
    wh2 = _blockdiag2(head_w)
    wf2 = _blockdiag2(fc_w)

    out = pl.pallas_call(
        functools.partial(_fused_net_kernel, Ho=Ho, Wo=Wo),
        out_shape=jax.ShapeDtypeStruct((P, 1, 2 * NC), jnp.float32),
        grid=(P,),
        in_specs=[
            pl.BlockSpec((1, S, 9 * 2 * C_IN), lambda b: (b, 0, 0)),
            pl.BlockSpec((9 * 2 * C_IN, 2 * Cs), lambda b: (0, 0)),
            pl.BlockSpec((1, 2 * Cs), lambda b: (0, 0)),
            pl.BlockSpec((2 * Cs, 2 * C), lambda b: (0, 0)),
            pl.BlockSpec((1, 2 * C), lambda b: (0, 0)),
            pl.BlockSpec((9, 2 * C), lambda b: (0, 0)),
            pl.BlockSpec((1, 2 * C), lambda b: (0, 0)),
            pl.BlockSpec((2 * C, 2 * Cse), lambda b: (0, 0)),
            pl.BlockSpec((1, 2 * Cse), lambda b: (0, 0)),
            pl.BlockSpec((2 * Cse, 2 * C), lambda b: (0, 0)),
            pl.BlockSpec((1, 2 * C), lambda b: (0, 0)),
            pl.BlockSpec((2 * C, 2 * Cs), lambda b: (0, 0)),
            pl.BlockSpec((1, 2 * Cs), lambda b: (0, 0)),
            pl.BlockSpec((2 * Cs, 2 * Ch), lambda b: (0, 0)),
            pl.BlockSpec((1, 2 * Ch), lambda b: (0, 0)),
            pl.BlockSpec((2 * Ch, 2 * NC), lambda b: (0, 0)),
            pl.BlockSpec((1, 2 * NC), lambda b: (0, 0)),
        ],
        out_specs=pl.BlockSpec((1, 1, 2 * NC), lambda b: (b, 0, 0)),
        compiler_params=pltpu.CompilerParams(dimension_semantics=("parallel",)),
    )(cols, ws2, _pair2(stem_b), we2, _pair2(exp_b),
      wdw2, _pair2(dw_b), w1_2, _pair2(se_r_b), w2_2, _pair2(se_e_b),
      wp2, _pair2(proj_b), wh2, _pair2(head_b), wf2, _pair2(fc_b))
    return out.reshape(P, 2, NC).reshape(B, NC)
